# baseline (speedup 1.0000x reference)
import functools

import jax
import jax.numpy as jnp
from jax import lax
from jax.experimental import pallas as pl
from jax.experimental.pallas import tpu as pltpu
from jax.experimental.pallas import tpu_sc as plsc

N = 10000
E = 320000
D = 128
R = 8

NC = 2
NS = 16
NW = NC * NS
EPW = E // NW
CH = 80
NCH = EPW // CH
CHA = 40
NCHA = EPW // CHA
KB = 5
RN = R * N
RN_PAD = 81920
CNT_SLICE = RN_PAD // NS
RPT = N // NS

_mesh = plsc.VectorSubcoreMesh(core_axis_name="c", subcore_axis_name="s")
_sc_params = pltpu.CompilerParams(
    needs_layout_passes=False, use_tc_tiling_on_sc=False)


BN = 400
NB = N // BN
DC = (R + 1) * D


def _mm_body(x_ref, w_ref, b_ref, out_ref, base_ref):
    res = (jnp.dot(x_ref[...], w_ref[...], preferred_element_type=jnp.float32)
           + b_ref[...])
    out_ref[...] = res.astype(jnp.bfloat16)
    base_ref[...] = res[:, R * D:]


def _mm_relu_body(a_ref, w_ref, b_ref, out_ref, base_ref):
    xb = jnp.maximum(a_ref[0] + a_ref[1], 0.0)
    res = (jnp.dot(xb, w_ref[...], preferred_element_type=jnp.float32)
           + b_ref[...])
    out_ref[...] = res.astype(jnp.bfloat16)
    base_ref[...] = res[:, R * D:]


def _matmul_all(x, wcat, bcat):
    return pl.pallas_call(
        _mm_body,
        grid=(NB,),
        in_specs=[
            pl.BlockSpec((BN, D), lambda i: (i, 0)),
            pl.BlockSpec((D, DC), lambda i: (0, 0)),
            pl.BlockSpec((1, DC), lambda i: (0, 0)),
        ],
        out_specs=[pl.BlockSpec((BN, DC), lambda i: (i, 0)),
                   pl.BlockSpec((BN, D), lambda i: (i, 0))],
        out_shape=[jax.ShapeDtypeStruct((N, DC), jnp.bfloat16),
                   jax.ShapeDtypeStruct((N, D), jnp.float32)],
    )(x, wcat, bcat[None])


def _matmul_all_relu(acc, wcat, bcat):
    return pl.pallas_call(
        _mm_relu_body,
        grid=(NB,),
        in_specs=[
            pl.BlockSpec((NC, BN, D), lambda i: (0, i, 0)),
            pl.BlockSpec((D, DC), lambda i: (0, 0)),
            pl.BlockSpec((1, DC), lambda i: (0, 0)),
        ],
        out_specs=[pl.BlockSpec((BN, DC), lambda i: (i, 0)),
                   pl.BlockSpec((BN, D), lambda i: (i, 0))],
        out_shape=[jax.ShapeDtypeStruct((N, DC), jnp.bfloat16),
                   jax.ShapeDtypeStruct((N, D), jnp.float32)],
    )(acc, wcat, bcat[None])


def _idx_body(src_ref, dst_ref, et_ref, fs_ref, fd_ref):
    fs_ref[...] = src_ref[...] * (R + 1) + et_ref[...]
    fd_ref[...] = et_ref[...] * N + dst_ref[...]


def _idx_prep(src2, dst2, et2):
    rows = E // D
    return pl.pallas_call(
        _idx_body,
        out_shape=[jax.ShapeDtypeStruct((rows, D), jnp.int32)] * 2,
    )(src2, dst2, et2)


def _sum_body(a_ref, out_ref):
    out_ref[...] = a_ref[0] + a_ref[1]


def _final_sum(acc):
    return pl.pallas_call(
        _sum_body,
        grid=(NB,),
        in_specs=[pl.BlockSpec((NC, BN, D), lambda i: (0, i, 0))],
        out_specs=pl.BlockSpec((BN, D), lambda i: (i, 0)),
        out_shape=jax.ShapeDtypeStruct((N, D), jnp.float32),
    )(acc)


@functools.partial(
    pl.kernel,
    out_type=jax.ShapeDtypeStruct((NC, RN_PAD), jnp.float32),
    mesh=_mesh,
    compiler_params=_sc_params,
    scratch_types=[
        pltpu.VMEM_SHARED((RN_PAD,), jnp.float32),
        pltpu.VMEM((CNT_SLICE,), jnp.float32),
        pltpu.VMEM((CH,), jnp.float32),
        pltpu.VMEM((NCH, CH), jnp.int32),
        pltpu.SemaphoreType.DMA,
        pltpu.SemaphoreType.DMA,
    ],
)
def _count_kernel(fd3_hbm, cnt_hbm, cnt_sh, zbuf, ones_v, fdi_v, psem, ssem):
    c = lax.axis_index("c")
    s = lax.axis_index("s")
    wid = s * NC + c
    off = s * CNT_SLICE

    pltpu.async_copy(fd3_hbm.at[wid], fdi_v, psem)

    def z16(i, _):
        zbuf[pl.ds(i * 16, 16)] = jnp.zeros((16,), jnp.float32)
        return 0
    lax.fori_loop(0, CNT_SLICE // 16, z16, 0)
    pltpu.sync_copy(zbuf, cnt_sh.at[pl.ds(off, CNT_SLICE)])

    def o16(i, _):
        ones_v[pl.ds(i * 16, 16)] = jnp.ones((16,), jnp.float32)
        return 0
    lax.fori_loop(0, CH // 16, o16, 0)
    pltpu.make_async_copy(fd3_hbm.at[wid], fdi_v, psem).wait()
    plsc.subcore_barrier()

    GRP = 8

    def grp(i, _):
        for j in range(GRP):
            pltpu.async_copy(ones_v, cnt_sh.at[fdi_v.at[i * GRP + j]],
                             ssem, add=True)
        for j in range(GRP):
            pltpu.make_async_copy(ones_v, cnt_sh.at[fdi_v.at[0]], ssem).wait()
        return 0
    lax.fori_loop(0, NCH // GRP, grp, 0)
    for t in range((NCH // GRP) * GRP, NCH):
        pltpu.async_copy(ones_v, cnt_sh.at[fdi_v.at[t]], ssem, add=True)
    for t in range((NCH // GRP) * GRP, NCH):
        pltpu.make_async_copy(ones_v, cnt_sh.at[fdi_v.at[0]], ssem).wait()
    plsc.subcore_barrier()

    pltpu.sync_copy(cnt_sh.at[pl.ds(off, CNT_SLICE)],
                    cnt_hbm.at[c, pl.ds(off, CNT_SLICE)])


@functools.partial(
    pl.kernel,
    out_type=jax.ShapeDtypeStruct((NW, NCH, CH), jnp.float32),
    mesh=_mesh,
    compiler_params=_sc_params,
    scratch_types=[
        pltpu.VMEM_SHARED((RN_PAD,), jnp.float32),
        pltpu.VMEM((CNT_SLICE,), jnp.float32),
        pltpu.VMEM((CNT_SLICE,), jnp.float32),
        pltpu.VMEM((RN,), jnp.float32),
        pltpu.VMEM((NCH, CH), jnp.int32),
        pltpu.VMEM((NCH, CH), jnp.float32),
        pltpu.SemaphoreType.DMA,
    ],
)
def _wgt_kernel(cnt_hbm, fd3_hbm, w3_hbm, inv_sh, c0_v, c1_v, inv_v, fdi_v,
                wo_v, psem):
    c = lax.axis_index("c")
    s = lax.axis_index("s")
    wid = s * NC + c
    off = s * CNT_SLICE

    pltpu.async_copy(fd3_hbm.at[wid], fdi_v, psem)
    pltpu.sync_copy(cnt_hbm.at[0, pl.ds(off, CNT_SLICE)], c0_v)
    pltpu.sync_copy(cnt_hbm.at[1, pl.ds(off, CNT_SLICE)], c1_v)

    def inv16(i, _):
        v = c0_v[pl.ds(i * 16, 16)] + c1_v[pl.ds(i * 16, 16)]
        c0_v[pl.ds(i * 16, 16)] = 1.0 / jnp.maximum(v, 1.0)
        return 0
    lax.fori_loop(0, CNT_SLICE // 16, inv16, 0)
    pltpu.sync_copy(c0_v, inv_sh.at[pl.ds(off, CNT_SLICE)])
    plsc.subcore_barrier()

    pltpu.sync_copy(inv_sh.at[pl.ds(0, RN)], inv_v)
    pltpu.make_async_copy(fd3_hbm.at[wid], fdi_v, psem).wait()

    def wrow(g, _):
        for j in range(CH // 16):
            ii = fdi_v[g, pl.ds(j * 16, 16)]
            wo_v[g, pl.ds(j * 16, 16)] = plsc.load_gather(inv_v, [ii])
        return 0
    lax.fori_loop(0, NCH, wrow, 0)
    pltpu.sync_copy(wo_v, w3_hbm.at[wid])


@functools.partial(
    pl.kernel,
    out_type=jax.ShapeDtypeStruct((NC, N, D), jnp.float32),
    mesh=_mesh,
    compiler_params=_sc_params,
    scratch_types=(
        [pltpu.VMEM_SHARED((N, D), jnp.float32)]
        + [pltpu.VMEM((NCHA, CHA), jnp.int32)]
        + [pltpu.VMEM((CHA, D // 2), jnp.int32)] * KB
        + [pltpu.VMEM((CHA, D), jnp.float32)] * KB
        + [pltpu.VMEM((CHA,), jnp.int32)] * KB
        + [pltpu.VMEM((CHA,), jnp.float32)] * KB
        + [pltpu.SemaphoreType.DMA]
        + [pltpu.SemaphoreType.DMA] * KB
        + [pltpu.SemaphoreType.DMA] * KB
    ),
)
def _agg_kernel(hflat_hbm, base_hbm, zeros_hbm, fs3_hbm, dst3_hbm, w3_hbm,
                out_hbm, acc_sh, si_v, *slots):
    raw = slots[0:KB]
    rows = slots[KB:2 * KB]
    dib = slots[2 * KB:3 * KB]
    wb = slots[3 * KB:4 * KB]
    psem = slots[4 * KB]
    gsems = slots[4 * KB + 1:5 * KB + 1]
    ssems = slots[5 * KB + 1:6 * KB + 1]

    c = lax.axis_index("c")
    s = lax.axis_index("s")
    wid = s * NC + c
    row0 = s * RPT

    pltpu.async_copy(fs3_hbm.at[wid], si_v, psem)

    @pl.when(c == 0)
    def _():
        pltpu.async_copy(base_hbm.at[pl.ds(row0, RPT)],
                         acc_sh.at[pl.ds(row0, RPT)], psem)

    @pl.when(c == 1)
    def _():
        pltpu.async_copy(zeros_hbm.at[pl.ds(row0, RPT)],
                         acc_sh.at[pl.ds(row0, RPT)], psem)
    pltpu.make_async_copy(fs3_hbm.at[wid], si_v, psem).wait()
    pltpu.make_async_copy(zeros_hbm.at[pl.ds(row0, RPT)],
                          acc_sh.at[pl.ds(row0, RPT)], psem).wait()
    plsc.subcore_barrier()

    def fetch(t, b):
        pltpu.async_copy(dst3_hbm.at[wid, t], dib[b], gsems[b])
        pltpu.async_copy(w3_hbm.at[wid, t], wb[b], gsems[b])
        pltpu.async_copy(hflat_hbm.at[si_v.at[t]], raw[b], gsems[b])

    def fwait(t, b):
        pltpu.make_async_copy(dst3_hbm.at[wid, t], dib[b], gsems[b]).wait()
        pltpu.make_async_copy(w3_hbm.at[wid, t], wb[b], gsems[b]).wait()
        pltpu.make_async_copy(hflat_hbm.at[si_v.at[0]], raw[b],
                              gsems[b]).wait()

    def scat(b):
        pltpu.async_copy(rows[b], acc_sh.at[dib[b]], ssems[b], add=True)

    def swait(b):
        pltpu.make_async_copy(rows[b], acc_sh.at[dib[b]], ssems[b]).wait()

    def scale(b):
        himask = jnp.full((16,), -65536, jnp.int32)
        shift16 = jnp.full((16,), 16, jnp.int32)

        def body(j, _):
            ws = plsc.load_gather(wb[b], [lax.broadcast(j, (16,))])
            rv = raw[b]
            r = rows[b]
            for g4 in range(D // 32):
                v = rv[j, pl.ds(g4 * 16, 16)]
                lo = plsc.bitcast(lax.shift_left(v, shift16), jnp.float32)
                hi = plsc.bitcast(v & himask, jnp.float32)
                r[j, pl.ds(g4 * 32, 16)] = lo * ws
                r[j, pl.ds(g4 * 32 + 16, 16)] = hi * ws
            return 0
        lax.fori_loop(0, CHA, body, 0)

    def step(t, b):
        fwait(t, b)
        scale(b)
        scat(b)
        bb = (b + KB - 1) % KB

        @pl.when(t >= 1)
        def _():
            swait(bb)

        @pl.when(t + KB - 1 < NCHA)
        def _():
            fetch(t + KB - 1, bb)

    for u in range(KB - 1):
        fetch(jnp.int32(u), u)

    def group(i, _):
        for k in range(KB):
            step(i * KB + k, k)
        return 0
    lax.fori_loop(0, NCHA // KB, group, 0)
    swait((NCHA - 1) % KB)

    plsc.subcore_barrier()
    pltpu.sync_copy(acc_sh.at[pl.ds(row0, RPT)],
                    out_hbm.at[c, pl.ds(row0, RPT)])


def kernel(x, edge_index, edge_type, W_rel1, W_root1, b1, W_rel2, W_root2, b2):
    rows = E // D
    src2 = edge_index[0].reshape(rows, D)
    dst2 = edge_index[1].reshape(rows, D)
    et2 = edge_type.reshape(rows, D)
    fs, fd = _idx_prep(src2, dst2, et2)
    fs3 = fs.reshape(NW, NCHA, CHA)
    fd3 = fd.reshape(NW, NCH, CH)
    dst3 = edge_index[1].reshape(NW, NCHA, CHA)

    cnt = _count_kernel(fd3)
    w3 = _wgt_kernel(cnt, fd3).reshape(NW, NCHA, CHA)

    zeros_nd = jnp.zeros((N, D), jnp.float32)

    def _wcat(w_rel, w_root):
        wr = jnp.transpose(w_rel, (1, 0, 2)).reshape(D, R * 4, 2, 16)
        wr = wr.swapaxes(2, 3).reshape(D, R * D)
        return jnp.concatenate([wr, w_root], axis=1)

    def _packed(h):
        return jax.lax.bitcast_convert_type(
            h.reshape(N * (R + 1), D // 2, 2), jnp.int32)

    wcat1 = _wcat(W_rel1, W_root1)
    bcat1 = jnp.concatenate([jnp.zeros((R * D,), jnp.float32), b1])
    h1, base1 = _matmul_all(x, wcat1, bcat1)
    acc1 = _agg_kernel(_packed(h1), base1, zeros_nd, fs3, dst3, w3)

    wcat2 = _wcat(W_rel2, W_root2)
    bcat2 = jnp.concatenate([jnp.zeros((R * D,), jnp.float32), b2])
    h2, base2 = _matmul_all_relu(acc1, wcat2, bcat2)
    acc2 = _agg_kernel(_packed(h2), base2, zeros_nd, fs3, dst3, w3)

    return _final_sum(acc2)


# speedup vs baseline: 12.5544x; 12.5544x over previous
import functools

import jax
import jax.numpy as jnp
from jax import lax
from jax.experimental import pallas as pl
from jax.experimental.pallas import tpu as pltpu
from jax.experimental.pallas import tpu_sc as plsc

N = 10000
E = 320000
D = 128
R = 8

NC = 2
NS = 16
NW = NC * NS
EPW = E // NW
CH = 80
NCH = EPW // CH
CHA = 40
NCHA = EPW // CHA
KB = 5
RN = R * N
RN_PAD = 81920
CNT_SLICE = RN_PAD // NS
RPT = N // NS

_mesh = plsc.VectorSubcoreMesh(core_axis_name="c", subcore_axis_name="s")
_sc_params = pltpu.CompilerParams(
    needs_layout_passes=False, use_tc_tiling_on_sc=False)


BN = 400
NB = N // BN
DC = (R + 1) * D


def _mm_body(x_ref, w_ref, b_ref, out_ref, base_ref):
    res = (jnp.dot(x_ref[...], w_ref[...], preferred_element_type=jnp.float32)
           + b_ref[...])
    out_ref[...] = res.astype(jnp.bfloat16)
    base_ref[...] = res[:, R * D:]


def _mm_relu_body(a_ref, w_ref, b_ref, out_ref, base_ref):
    xb = jnp.maximum(a_ref[0] + a_ref[1], 0.0)
    res = (jnp.dot(xb, w_ref[...], preferred_element_type=jnp.float32)
           + b_ref[...])
    out_ref[...] = res.astype(jnp.bfloat16)
    base_ref[...] = res[:, R * D:]


def _matmul_all(x, wcat, bcat):
    return pl.pallas_call(
        _mm_body,
        grid=(NB,),
        in_specs=[
            pl.BlockSpec((BN, D), lambda i: (i, 0)),
            pl.BlockSpec((D, DC), lambda i: (0, 0)),
            pl.BlockSpec((1, DC), lambda i: (0, 0)),
        ],
        out_specs=[pl.BlockSpec((BN, DC), lambda i: (i, 0)),
                   pl.BlockSpec((BN, D), lambda i: (i, 0))],
        out_shape=[jax.ShapeDtypeStruct((N, DC), jnp.bfloat16),
                   jax.ShapeDtypeStruct((N, D), jnp.float32)],
    )(x, wcat, bcat[None])


def _matmul_all_relu(acc, wcat, bcat):
    return pl.pallas_call(
        _mm_relu_body,
        grid=(NB,),
        in_specs=[
            pl.BlockSpec((NC, BN, D), lambda i: (0, i, 0)),
            pl.BlockSpec((D, DC), lambda i: (0, 0)),
            pl.BlockSpec((1, DC), lambda i: (0, 0)),
        ],
        out_specs=[pl.BlockSpec((BN, DC), lambda i: (i, 0)),
                   pl.BlockSpec((BN, D), lambda i: (i, 0))],
        out_shape=[jax.ShapeDtypeStruct((N, DC), jnp.bfloat16),
                   jax.ShapeDtypeStruct((N, D), jnp.float32)],
    )(acc, wcat, bcat[None])


def _idx_body(src_ref, dst_ref, et_ref, fs_ref, fd_ref):
    fs_ref[...] = src_ref[...] * (R + 1) + et_ref[...]
    fd_ref[...] = et_ref[...] * N + dst_ref[...]


def _idx_prep(src2, dst2, et2):
    rows = E // D
    return pl.pallas_call(
        _idx_body,
        out_shape=[jax.ShapeDtypeStruct((rows, D), jnp.int32)] * 2,
    )(src2, dst2, et2)


def _sum_body(a_ref, out_ref):
    out_ref[...] = a_ref[0] + a_ref[1]


def _final_sum(acc):
    return pl.pallas_call(
        _sum_body,
        grid=(NB,),
        in_specs=[pl.BlockSpec((NC, BN, D), lambda i: (0, i, 0))],
        out_specs=pl.BlockSpec((BN, D), lambda i: (i, 0)),
        out_shape=jax.ShapeDtypeStruct((N, D), jnp.float32),
    )(acc)


@functools.partial(
    pl.kernel,
    out_type=jax.ShapeDtypeStruct((NC, RN_PAD), jnp.float32),
    mesh=_mesh,
    compiler_params=_sc_params,
    scratch_types=[
        pltpu.VMEM_SHARED((RN_PAD,), jnp.float32),
        pltpu.VMEM((CNT_SLICE,), jnp.float32),
        pltpu.VMEM((CH,), jnp.float32),
        pltpu.VMEM((NCH, CH), jnp.int32),
        pltpu.SemaphoreType.DMA,
        pltpu.SemaphoreType.DMA,
    ],
)
def _count_kernel(fd3_hbm, cnt_hbm, cnt_sh, zbuf, ones_v, fdi_v, psem, ssem):
    c = lax.axis_index("c")
    s = lax.axis_index("s")
    wid = s * NC + c
    off = s * CNT_SLICE

    pltpu.async_copy(fd3_hbm.at[wid], fdi_v, psem)

    def z16(i, _):
        zbuf[pl.ds(i * 16, 16)] = jnp.zeros((16,), jnp.float32)
        return 0
    lax.fori_loop(0, CNT_SLICE // 16, z16, 0)
    pltpu.sync_copy(zbuf, cnt_sh.at[pl.ds(off, CNT_SLICE)])

    def o16(i, _):
        ones_v[pl.ds(i * 16, 16)] = jnp.ones((16,), jnp.float32)
        return 0
    lax.fori_loop(0, CH // 16, o16, 0)
    pltpu.make_async_copy(fd3_hbm.at[wid], fdi_v, psem).wait()
    plsc.subcore_barrier()

    GRP = 8

    def grp(i, _):
        for j in range(GRP):
            pltpu.async_copy(ones_v, cnt_sh.at[fdi_v.at[i * GRP + j]],
                             ssem, add=True)
        for j in range(GRP):
            pltpu.make_async_copy(ones_v, cnt_sh.at[fdi_v.at[0]], ssem).wait()
        return 0
    lax.fori_loop(0, NCH // GRP, grp, 0)
    for t in range((NCH // GRP) * GRP, NCH):
        pltpu.async_copy(ones_v, cnt_sh.at[fdi_v.at[t]], ssem, add=True)
    for t in range((NCH // GRP) * GRP, NCH):
        pltpu.make_async_copy(ones_v, cnt_sh.at[fdi_v.at[0]], ssem).wait()
    plsc.subcore_barrier()

    pltpu.sync_copy(cnt_sh.at[pl.ds(off, CNT_SLICE)],
                    cnt_hbm.at[c, pl.ds(off, CNT_SLICE)])


@functools.partial(
    pl.kernel,
    out_type=jax.ShapeDtypeStruct((NW, NCH, CH), jnp.float32),
    mesh=_mesh,
    compiler_params=_sc_params,
    scratch_types=[
        pltpu.VMEM_SHARED((RN_PAD,), jnp.float32),
        pltpu.VMEM((CNT_SLICE,), jnp.float32),
        pltpu.VMEM((CNT_SLICE,), jnp.float32),
        pltpu.VMEM((RN,), jnp.float32),
        pltpu.VMEM((NCH, CH), jnp.int32),
        pltpu.VMEM((NCH, CH), jnp.float32),
        pltpu.SemaphoreType.DMA,
    ],
)
def _wgt_kernel(cnt_hbm, fd3_hbm, w3_hbm, inv_sh, c0_v, c1_v, inv_v, fdi_v,
                wo_v, psem):
    c = lax.axis_index("c")
    s = lax.axis_index("s")
    wid = s * NC + c
    off = s * CNT_SLICE

    pltpu.async_copy(fd3_hbm.at[wid], fdi_v, psem)
    pltpu.sync_copy(cnt_hbm.at[0, pl.ds(off, CNT_SLICE)], c0_v)
    pltpu.sync_copy(cnt_hbm.at[1, pl.ds(off, CNT_SLICE)], c1_v)

    def inv16(i, _):
        v = c0_v[pl.ds(i * 16, 16)] + c1_v[pl.ds(i * 16, 16)]
        c0_v[pl.ds(i * 16, 16)] = 1.0 / jnp.maximum(v, 1.0)
        return 0
    lax.fori_loop(0, CNT_SLICE // 16, inv16, 0)
    pltpu.sync_copy(c0_v, inv_sh.at[pl.ds(off, CNT_SLICE)])
    plsc.subcore_barrier()

    pltpu.sync_copy(inv_sh.at[pl.ds(0, RN)], inv_v)
    pltpu.make_async_copy(fd3_hbm.at[wid], fdi_v, psem).wait()

    def wrow(g, _):
        for j in range(CH // 16):
            ii = fdi_v[g, pl.ds(j * 16, 16)]
            wo_v[g, pl.ds(j * 16, 16)] = plsc.load_gather(inv_v, [ii])
        return 0
    lax.fori_loop(0, NCH, wrow, 0)
    pltpu.sync_copy(wo_v, w3_hbm.at[wid])


@functools.partial(
    pl.kernel,
    out_type=jax.ShapeDtypeStruct((NC, N, D), jnp.float32),
    mesh=_mesh,
    compiler_params=_sc_params,
    scratch_types=(
        [pltpu.VMEM_SHARED((N, D), jnp.float32)]
        + [pltpu.VMEM((NCHA, CHA), jnp.int32)]
        + [pltpu.VMEM((CHA, D), jnp.bfloat16)] * KB
        + [pltpu.VMEM((CHA, D), jnp.float32)] * KB
        + [pltpu.VMEM((CHA,), jnp.int32)] * KB
        + [pltpu.VMEM((CHA,), jnp.float32)] * KB
        + [pltpu.SemaphoreType.DMA]
        + [pltpu.SemaphoreType.DMA] * KB
        + [pltpu.SemaphoreType.DMA] * KB
    ),
)
def _agg_kernel(hflat_hbm, base_hbm, zeros_hbm, fs3_hbm, dst3_hbm, w3_hbm,
                out_hbm, acc_sh, si_v, *slots):
    raw = slots[0:KB]
    rows = slots[KB:2 * KB]
    dib = slots[2 * KB:3 * KB]
    wb = slots[3 * KB:4 * KB]
    psem = slots[4 * KB]
    gsems = slots[4 * KB + 1:5 * KB + 1]
    ssems = slots[5 * KB + 1:6 * KB + 1]

    c = lax.axis_index("c")
    s = lax.axis_index("s")
    wid = s * NC + c
    row0 = s * RPT

    pltpu.async_copy(fs3_hbm.at[wid], si_v, psem)

    @pl.when(c == 0)
    def _():
        pltpu.async_copy(base_hbm.at[pl.ds(row0, RPT)],
                         acc_sh.at[pl.ds(row0, RPT)], psem)

    @pl.when(c == 1)
    def _():
        pltpu.async_copy(zeros_hbm.at[pl.ds(row0, RPT)],
                         acc_sh.at[pl.ds(row0, RPT)], psem)
    pltpu.make_async_copy(fs3_hbm.at[wid], si_v, psem).wait()
    pltpu.make_async_copy(zeros_hbm.at[pl.ds(row0, RPT)],
                          acc_sh.at[pl.ds(row0, RPT)], psem).wait()
    plsc.subcore_barrier()

    def fetch(t, b):
        pltpu.async_copy(dst3_hbm.at[wid, t], dib[b], gsems[b])
        pltpu.async_copy(w3_hbm.at[wid, t], wb[b], gsems[b])
        pltpu.async_copy(hflat_hbm.at[si_v.at[t]], raw[b], gsems[b])

    def fwait(t, b):
        pltpu.make_async_copy(dst3_hbm.at[wid, t], dib[b], gsems[b]).wait()
        pltpu.make_async_copy(w3_hbm.at[wid, t], wb[b], gsems[b]).wait()
        pltpu.make_async_copy(hflat_hbm.at[si_v.at[0]], raw[b],
                              gsems[b]).wait()

    def scat(b):
        pltpu.async_copy(rows[b], acc_sh.at[dib[b]], ssems[b], add=True)

    def swait(b):
        pltpu.make_async_copy(rows[b], acc_sh.at[dib[b]], ssems[b]).wait()

    def scale(b):
        himask = jnp.full((16,), -65536, jnp.int32)
        shift16 = jnp.full((16,), 16, jnp.int32)

        def body(j, _):
            ws = plsc.load_gather(wb[b], [lax.broadcast(j, (16,))])
            rv = raw[b]
            r = rows[b]
            for g4 in range(D // 32):
                v = plsc.bitcast(rv[j, pl.ds(g4 * 32, 32)], jnp.int32)
                lo = plsc.bitcast(lax.shift_left(v, shift16), jnp.float32)
                hi = plsc.bitcast(v & himask, jnp.float32)
                r[j, pl.ds(g4 * 32, 16)] = lo * ws
                r[j, pl.ds(g4 * 32 + 16, 16)] = hi * ws
            return 0
        lax.fori_loop(0, CHA, body, 0)

    def step(t, b):
        fwait(t, b)
        scale(b)
        scat(b)
        bb = (b + KB - 1) % KB

        @pl.when(t >= 1)
        def _():
            swait(bb)

        @pl.when(t + KB - 1 < NCHA)
        def _():
            fetch(t + KB - 1, bb)

    for u in range(KB - 1):
        fetch(jnp.int32(u), u)

    def group(i, _):
        for k in range(KB):
            step(i * KB + k, k)
        return 0
    lax.fori_loop(0, NCHA // KB, group, 0)
    swait((NCHA - 1) % KB)

    plsc.subcore_barrier()
    pltpu.sync_copy(acc_sh.at[pl.ds(row0, RPT)],
                    out_hbm.at[c, pl.ds(row0, RPT)])


def kernel(x, edge_index, edge_type, W_rel1, W_root1, b1, W_rel2, W_root2, b2):
    rows = E // D
    src2 = edge_index[0].reshape(rows, D)
    dst2 = edge_index[1].reshape(rows, D)
    et2 = edge_type.reshape(rows, D)
    fs, fd = _idx_prep(src2, dst2, et2)
    fs3 = fs.reshape(NW, NCHA, CHA)
    fd3 = fd.reshape(NW, NCH, CH)
    dst3 = edge_index[1].reshape(NW, NCHA, CHA)

    cnt = _count_kernel(fd3)
    w3 = _wgt_kernel(cnt, fd3).reshape(NW, NCHA, CHA)

    zeros_nd = jnp.zeros((N, D), jnp.float32)

    def _wcat(w_rel, w_root):
        wr = jnp.transpose(w_rel, (1, 0, 2)).reshape(D, R * 4, 2, 16)
        wr = wr.swapaxes(2, 3).reshape(D, R * D)
        return jnp.concatenate([wr, w_root], axis=1)

    def _packed(h):
        return h.reshape(N * (R + 1), D)

    wcat1 = _wcat(W_rel1, W_root1)
    bcat1 = jnp.concatenate([jnp.zeros((R * D,), jnp.float32), b1])
    h1, base1 = _matmul_all(x, wcat1, bcat1)
    acc1 = _agg_kernel(_packed(h1), base1, zeros_nd, fs3, dst3, w3)

    wcat2 = _wcat(W_rel2, W_root2)
    bcat2 = jnp.concatenate([jnp.zeros((R * D,), jnp.float32), b2])
    h2, base2 = _matmul_all_relu(acc1, wcat2, bcat2)
    acc2 = _agg_kernel(_packed(h2), base2, zeros_nd, fs3, dst3, w3)

    return _final_sum(acc2)


# speedup vs baseline: 22.0248x; 1.7544x over previous
import functools

import jax
import jax.numpy as jnp
from jax import lax
from jax.experimental import pallas as pl
from jax.experimental.pallas import tpu as pltpu
from jax.experimental.pallas import tpu_sc as plsc

N = 10000
E = 320000
D = 128
R = 8

NC = 2
NS = 16
NW = NC * NS
EPW = E // NW
CH = 80
NCH = EPW // CH
CHA = 40
NCHA = EPW // CHA
KB = 5
RN = R * N
RN_PAD = 81920
CNT_SLICE = RN_PAD // NS
RPT = N // NS

_mesh = plsc.VectorSubcoreMesh(core_axis_name="c", subcore_axis_name="s")
_sc_params = pltpu.CompilerParams(
    needs_layout_passes=False, use_tc_tiling_on_sc=False)


BN = 400
NB = N // BN
DC = (R + 1) * D


def _mm_body(x_ref, w_ref, b_ref, out_ref, base_ref):
    res = (jnp.dot(x_ref[...], w_ref[...], preferred_element_type=jnp.float32)
           + b_ref[...])
    out_ref[...] = res
    base_ref[...] = res[:, R * D:]


def _mm_relu_body(a_ref, w_ref, b_ref, out_ref, base_ref):
    xb = jnp.maximum(a_ref[0] + a_ref[1], 0.0)
    res = (jnp.dot(xb, w_ref[...], preferred_element_type=jnp.float32)
           + b_ref[...])
    out_ref[...] = res
    base_ref[...] = res[:, R * D:]


def _matmul_all(x, wcat, bcat):
    return pl.pallas_call(
        _mm_body,
        grid=(NB,),
        in_specs=[
            pl.BlockSpec((BN, D), lambda i: (i, 0)),
            pl.BlockSpec((D, DC), lambda i: (0, 0)),
            pl.BlockSpec((1, DC), lambda i: (0, 0)),
        ],
        out_specs=[pl.BlockSpec((BN, DC), lambda i: (i, 0)),
                   pl.BlockSpec((BN, D), lambda i: (i, 0))],
        out_shape=[jax.ShapeDtypeStruct((N, DC), jnp.float32),
                   jax.ShapeDtypeStruct((N, D), jnp.float32)],
    )(x, wcat, bcat[None])


def _matmul_all_relu(acc, wcat, bcat):
    return pl.pallas_call(
        _mm_relu_body,
        grid=(NB,),
        in_specs=[
            pl.BlockSpec((NC, BN, D), lambda i: (0, i, 0)),
            pl.BlockSpec((D, DC), lambda i: (0, 0)),
            pl.BlockSpec((1, DC), lambda i: (0, 0)),
        ],
        out_specs=[pl.BlockSpec((BN, DC), lambda i: (i, 0)),
                   pl.BlockSpec((BN, D), lambda i: (i, 0))],
        out_shape=[jax.ShapeDtypeStruct((N, DC), jnp.float32),
                   jax.ShapeDtypeStruct((N, D), jnp.float32)],
    )(acc, wcat, bcat[None])


def _idx_body(src_ref, dst_ref, et_ref, fs_ref, fd_ref):
    fs_ref[...] = src_ref[...] * (R + 1) + et_ref[...]
    fd_ref[...] = et_ref[...] * N + dst_ref[...]


def _idx_prep(src2, dst2, et2):
    rows = E // D
    return pl.pallas_call(
        _idx_body,
        out_shape=[jax.ShapeDtypeStruct((rows, D), jnp.int32)] * 2,
    )(src2, dst2, et2)


def _sum_body(a_ref, out_ref):
    out_ref[...] = a_ref[0] + a_ref[1]


def _final_sum(acc):
    return pl.pallas_call(
        _sum_body,
        grid=(NB,),
        in_specs=[pl.BlockSpec((NC, BN, D), lambda i: (0, i, 0))],
        out_specs=pl.BlockSpec((BN, D), lambda i: (i, 0)),
        out_shape=jax.ShapeDtypeStruct((N, D), jnp.float32),
    )(acc)


@functools.partial(
    pl.kernel,
    out_type=jax.ShapeDtypeStruct((NC, RN_PAD), jnp.float32),
    mesh=_mesh,
    compiler_params=_sc_params,
    scratch_types=[
        pltpu.VMEM_SHARED((RN_PAD,), jnp.float32),
        pltpu.VMEM((CNT_SLICE,), jnp.float32),
        pltpu.VMEM((CH,), jnp.float32),
        pltpu.VMEM((NCH, CH), jnp.int32),
        pltpu.SemaphoreType.DMA,
        pltpu.SemaphoreType.DMA,
    ],
)
def _count_kernel(fd3_hbm, cnt_hbm, cnt_sh, zbuf, ones_v, fdi_v, psem, ssem):
    c = lax.axis_index("c")
    s = lax.axis_index("s")
    wid = s * NC + c
    off = s * CNT_SLICE

    pltpu.async_copy(fd3_hbm.at[wid], fdi_v, psem)

    def z16(i, _):
        zbuf[pl.ds(i * 16, 16)] = jnp.zeros((16,), jnp.float32)
        return 0
    lax.fori_loop(0, CNT_SLICE // 16, z16, 0)
    pltpu.sync_copy(zbuf, cnt_sh.at[pl.ds(off, CNT_SLICE)])

    def o16(i, _):
        ones_v[pl.ds(i * 16, 16)] = jnp.ones((16,), jnp.float32)
        return 0
    lax.fori_loop(0, CH // 16, o16, 0)
    pltpu.make_async_copy(fd3_hbm.at[wid], fdi_v, psem).wait()
    plsc.subcore_barrier()

    GRP = 8

    def grp(i, _):
        for j in range(GRP):
            pltpu.async_copy(ones_v, cnt_sh.at[fdi_v.at[i * GRP + j]],
                             ssem, add=True)
        for j in range(GRP):
            pltpu.make_async_copy(ones_v, cnt_sh.at[fdi_v.at[0]], ssem).wait()
        return 0
    lax.fori_loop(0, NCH // GRP, grp, 0)
    for t in range((NCH // GRP) * GRP, NCH):
        pltpu.async_copy(ones_v, cnt_sh.at[fdi_v.at[t]], ssem, add=True)
    for t in range((NCH // GRP) * GRP, NCH):
        pltpu.make_async_copy(ones_v, cnt_sh.at[fdi_v.at[0]], ssem).wait()
    plsc.subcore_barrier()

    pltpu.sync_copy(cnt_sh.at[pl.ds(off, CNT_SLICE)],
                    cnt_hbm.at[c, pl.ds(off, CNT_SLICE)])


@functools.partial(
    pl.kernel,
    out_type=jax.ShapeDtypeStruct((NW, NCH, CH), jnp.float32),
    mesh=_mesh,
    compiler_params=_sc_params,
    scratch_types=[
        pltpu.VMEM_SHARED((RN_PAD,), jnp.float32),
        pltpu.VMEM((CNT_SLICE,), jnp.float32),
        pltpu.VMEM((CNT_SLICE,), jnp.float32),
        pltpu.VMEM((RN,), jnp.float32),
        pltpu.VMEM((NCH, CH), jnp.int32),
        pltpu.VMEM((NCH, CH), jnp.float32),
        pltpu.SemaphoreType.DMA,
    ],
)
def _wgt_kernel(cnt_hbm, fd3_hbm, w3_hbm, inv_sh, c0_v, c1_v, inv_v, fdi_v,
                wo_v, psem):
    c = lax.axis_index("c")
    s = lax.axis_index("s")
    wid = s * NC + c
    off = s * CNT_SLICE

    pltpu.async_copy(fd3_hbm.at[wid], fdi_v, psem)
    pltpu.sync_copy(cnt_hbm.at[0, pl.ds(off, CNT_SLICE)], c0_v)
    pltpu.sync_copy(cnt_hbm.at[1, pl.ds(off, CNT_SLICE)], c1_v)

    def inv16(i, _):
        v = c0_v[pl.ds(i * 16, 16)] + c1_v[pl.ds(i * 16, 16)]
        c0_v[pl.ds(i * 16, 16)] = 1.0 / jnp.maximum(v, 1.0)
        return 0
    lax.fori_loop(0, CNT_SLICE // 16, inv16, 0)
    pltpu.sync_copy(c0_v, inv_sh.at[pl.ds(off, CNT_SLICE)])
    plsc.subcore_barrier()

    pltpu.sync_copy(inv_sh.at[pl.ds(0, RN)], inv_v)
    pltpu.make_async_copy(fd3_hbm.at[wid], fdi_v, psem).wait()

    def wrow(g, _):
        for j in range(CH // 16):
            ii = fdi_v[g, pl.ds(j * 16, 16)]
            wo_v[g, pl.ds(j * 16, 16)] = plsc.load_gather(inv_v, [ii])
        return 0
    lax.fori_loop(0, NCH, wrow, 0)
    pltpu.sync_copy(wo_v, w3_hbm.at[wid])


@functools.partial(
    pl.kernel,
    out_type=jax.ShapeDtypeStruct((NC, N, D), jnp.float32),
    mesh=_mesh,
    compiler_params=_sc_params,
    scratch_types=(
        [pltpu.VMEM_SHARED((N, D), jnp.float32)]
        + [pltpu.VMEM((NCHA, CHA), jnp.int32)]
        + [pltpu.VMEM((CHA, D), jnp.float32)] * KB
        + [pltpu.VMEM((CHA,), jnp.int32)] * KB
        + [pltpu.VMEM((CHA,), jnp.float32)] * KB
        + [pltpu.SemaphoreType.DMA]
        + [pltpu.SemaphoreType.DMA] * KB
        + [pltpu.SemaphoreType.DMA] * KB
    ),
)
def _agg_kernel(hflat_hbm, base_hbm, zeros_hbm, fs3_hbm, dst3_hbm, w3_hbm,
                out_hbm, acc_sh, si_v, *slots):
    rows = slots[0:KB]
    dib = slots[KB:2 * KB]
    wb = slots[2 * KB:3 * KB]
    psem = slots[3 * KB]
    gsems = slots[3 * KB + 1:4 * KB + 1]
    ssems = slots[4 * KB + 1:5 * KB + 1]

    c = lax.axis_index("c")
    s = lax.axis_index("s")
    wid = s * NC + c
    row0 = s * RPT

    pltpu.async_copy(fs3_hbm.at[wid], si_v, psem)

    @pl.when(c == 0)
    def _():
        pltpu.async_copy(base_hbm.at[pl.ds(row0, RPT)],
                         acc_sh.at[pl.ds(row0, RPT)], psem)

    @pl.when(c == 1)
    def _():
        pltpu.async_copy(zeros_hbm.at[pl.ds(row0, RPT)],
                         acc_sh.at[pl.ds(row0, RPT)], psem)
    pltpu.make_async_copy(fs3_hbm.at[wid], si_v, psem).wait()
    pltpu.make_async_copy(zeros_hbm.at[pl.ds(row0, RPT)],
                          acc_sh.at[pl.ds(row0, RPT)], psem).wait()
    plsc.subcore_barrier()

    def fetch(t, b):
        pltpu.async_copy(dst3_hbm.at[wid, t], dib[b], gsems[b])
        pltpu.async_copy(w3_hbm.at[wid, t], wb[b], gsems[b])
        pltpu.async_copy(hflat_hbm.at[si_v.at[t]], rows[b], gsems[b])

    def fwait(t, b):
        pltpu.make_async_copy(dst3_hbm.at[wid, t], dib[b], gsems[b]).wait()
        pltpu.make_async_copy(w3_hbm.at[wid, t], wb[b], gsems[b]).wait()
        pltpu.make_async_copy(hflat_hbm.at[si_v.at[0]], rows[b],
                              gsems[b]).wait()

    def scat(b):
        pltpu.async_copy(rows[b], acc_sh.at[dib[b]], ssems[b], add=True)

    def swait(b):
        pltpu.make_async_copy(rows[b], acc_sh.at[dib[b]], ssems[b]).wait()

    def scale(b):
        def body(j, _):
            ws = plsc.load_gather(wb[b], [lax.broadcast(j, (16,))])
            r = rows[b]
            for cb in range(D // 16):
                r[j, pl.ds(cb * 16, 16)] = r[j, pl.ds(cb * 16, 16)] * ws
            return 0
        lax.fori_loop(0, CHA, body, 0)

    def step(t, b):
        fwait(t, b)
        scale(b)
        scat(b)
        bb = (b + KB - 1) % KB

        @pl.when(t >= 1)
        def _():
            swait(bb)

        @pl.when(t + KB - 1 < NCHA)
        def _():
            fetch(t + KB - 1, bb)

    for u in range(KB - 1):
        fetch(jnp.int32(u), u)

    def group(i, _):
        for k in range(KB):
            step(i * KB + k, k)
        return 0
    lax.fori_loop(0, NCHA // KB, group, 0)
    swait((NCHA - 1) % KB)

    plsc.subcore_barrier()
    pltpu.sync_copy(acc_sh.at[pl.ds(row0, RPT)],
                    out_hbm.at[c, pl.ds(row0, RPT)])


def kernel(x, edge_index, edge_type, W_rel1, W_root1, b1, W_rel2, W_root2, b2):
    rows = E // D
    src2 = edge_index[0].reshape(rows, D)
    dst2 = edge_index[1].reshape(rows, D)
    et2 = edge_type.reshape(rows, D)
    fs, fd = _idx_prep(src2, dst2, et2)
    fs3 = fs.reshape(NW, NCHA, CHA)
    fd3 = fd.reshape(NW, NCH, CH)
    dst3 = edge_index[1].reshape(NW, NCHA, CHA)

    cnt = _count_kernel(fd3)
    w3 = _wgt_kernel(cnt, fd3).reshape(NW, NCHA, CHA)

    zeros_nd = jnp.zeros((N, D), jnp.float32)

    wcat1 = jnp.concatenate(
        [jnp.transpose(W_rel1, (1, 0, 2)).reshape(D, R * D), W_root1], axis=1)
    bcat1 = jnp.concatenate([jnp.zeros((R * D,), jnp.float32), b1])
    h1, base1 = _matmul_all(x, wcat1, bcat1)
    acc1 = _agg_kernel(h1.reshape(N * (R + 1), D), base1,
                       zeros_nd, fs3, dst3, w3)

    wcat2 = jnp.concatenate(
        [jnp.transpose(W_rel2, (1, 0, 2)).reshape(D, R * D), W_root2], axis=1)
    bcat2 = jnp.concatenate([jnp.zeros((R * D,), jnp.float32), b2])
    h2, base2 = _matmul_all_relu(acc1, wcat2, bcat2)
    acc2 = _agg_kernel(h2.reshape(N * (R + 1), D), base2,
                       zeros_nd, fs3, dst3, w3)

    return _final_sum(acc2)
